# Initial kernel scaffold; baseline (speedup 1.0000x reference)
import jax
import jax.numpy as jnp
from jax.experimental import pallas as pl


def kernel(all_rois_left, all_rois_right, gt_boxes_left, gt_boxes_right, gt_dim_orien, gt_kpts, num_boxes):
    raise NotImplementedError("write your pallas kernel here")


# speedup vs baseline: 2.0744x; 2.0744x over previous
import jax
import jax.numpy as jnp
from jax.experimental import pallas as pl
from jax.experimental.pallas import tpu as pltpu

_NCLASSES = 4
_ROIS = 128
_FG = 32
_BG = _ROIS - _FG
_KPTS_GRID = 28
_LANES = 128
_ROWS = 157
_NP = _ROWS * _LANES


def _iou_terms(x1, y1, x2, y2, g, area_a):
    gx1, gy1, gx2, gy2 = g
    ix1 = jnp.maximum(x1, gx1)
    iy1 = jnp.maximum(y1, gy1)
    ix2 = jnp.minimum(x2, gx2)
    iy2 = jnp.minimum(y2, gy2)
    iw = jnp.maximum(ix2 - ix1 + 1.0, 0.0)
    ih = jnp.maximum(iy2 - iy1 + 1.0, 0.0)
    inter = iw * ih
    area_g = (gx2 - gx1 + 1.0) * (gy2 - gy1 + 1.0)
    return inter / (area_a + area_g - inter)


def _transform(ex1, ey1, ex2, ey2, gx1, gy1, gx2, gy2):
    ew = ex2 - ex1 + 1.0
    eh = ey2 - ey1 + 1.0
    ecx = ex1 + 0.5 * ew
    ecy = ey1 + 0.5 * eh
    gw = gx2 - gx1 + 1.0
    gh = gy2 - gy1 + 1.0
    gcx = gx1 + 0.5 * gw
    gcy = gy1 + 0.5 * gh
    dx = (gcx - ecx) / ew
    dy = (gcy - ecy) / eh
    dw = jnp.log(gw / ew)
    dh = jnp.log(gh / eh)
    return dx, dy, dw, dh


def _ptl_kernel(boxes_ref, gt_ref, nreal_ref, out_ref, assl_ref, idx_ref, flg_ref):
    flat_iota = (jax.lax.broadcasted_iota(jnp.int32, (_ROWS, _LANES), 0) * _LANES
                 + jax.lax.broadcasted_iota(jnp.int32, (_ROWS, _LANES), 1))
    lane_iota = jax.lax.broadcasted_iota(jnp.int32, (1, _LANES), 1)
    nreal = nreal_ref[0]

    x1l = boxes_ref[0]
    y1l = boxes_ref[1]
    x2l = boxes_ref[2]
    y2l = boxes_ref[3]
    x1r = boxes_ref[4]
    y1r = boxes_ref[5]
    x2r = boxes_ref[6]
    y2r = boxes_ref[7]
    area_l = (x2l - x1l + 1.0) * (y2l - y1l + 1.0)
    area_r = (x2r - x1r + 1.0) * (y2r - y1r + 1.0)

    max_l = jnp.full((_ROWS, _LANES), -1.0, jnp.float32)
    ass_l = jnp.zeros((_ROWS, _LANES), jnp.float32)
    max_r = jnp.full((_ROWS, _LANES), -1.0, jnp.float32)
    ass_r = jnp.zeros((_ROWS, _LANES), jnp.float32)
    for g in range(32):
        grow = gt_ref[g:g + 1, :]
        gl = (grow[:, 0:1], grow[:, 1:2], grow[:, 2:3], grow[:, 3:4])
        gr = (grow[:, 4:5], grow[:, 5:6], grow[:, 6:7], grow[:, 7:8])
        iou_l = _iou_terms(x1l, y1l, x2l, y2l, gl, area_l)
        iou_r = _iou_terms(x1r, y1r, x2r, y2r, gr, area_r)
        upd_l = iou_l > max_l
        ass_l = jnp.where(upd_l, jnp.float32(g), ass_l)
        max_l = jnp.where(upd_l, iou_l, max_l)
        upd_r = iou_r > max_r
        ass_r = jnp.where(upd_r, jnp.float32(g), ass_r)
        max_r = jnp.where(upd_r, iou_r, max_r)

    assl_ref[...] = ass_l

    valid = flat_iota < nreal
    fg_mask = (max_l >= 0.5) & (max_r >= 0.5) & (ass_l == ass_r)
    bg_mask = (max_l < 0.5) & (max_l >= 0.0) & jnp.logical_not(fg_mask)
    fg_key = jnp.where(valid, jnp.where(fg_mask, max_l, -1.0), -2.0)
    bg_key = jnp.where(valid, jnp.where(bg_mask, 1.0 - max_l, -1.0), -2.0)

    zero_row = jnp.zeros((1, _LANES), jnp.float32)

    def argmax_pick(key):
        m = jnp.max(key)
        eq = key == m
        idx = jnp.min(jnp.where(eq, flat_iota, _NP))
        return m, idx

    def dual_body(t, carry):
        fgk, bgk = carry
        mf, idf = argmax_pick(fgk)
        idx_ref[t] = idf
        flg_ref[t] = jnp.where(mf > 0.0, 1.0, 0.0)
        fgk = jnp.where(flat_iota == idf, -3.0, fgk)
        mb, idb = argmax_pick(bgk)
        idx_ref[_FG + t] = idb
        flg_ref[_FG + t] = 0.0
        bgk = jnp.where(flat_iota == idb, -3.0, bgk)
        return fgk, bgk

    def bg_body(t, bgk):
        mb, idb = argmax_pick(bgk)
        idx_ref[_FG + t] = idb
        flg_ref[_FG + t] = 0.0
        return jnp.where(flat_iota == idb, -3.0, bgk)

    _, bgk = jax.lax.fori_loop(0, _FG, dual_body, (fg_key, bg_key))
    jax.lax.fori_loop(_FG, _BG, bg_body, bgk)

    def gather_body(j, carry):
        comps, assv, actfg = carry
        idx = idx_ref[j]
        row = idx // _LANES
        lane = idx - row * _LANES
        onehot = lane_iota == lane
        new_comps = []
        for p in range(8):
            rv = boxes_ref[p, pl.ds(row, 1), :]
            v = jnp.sum(jnp.where(onehot, rv, 0.0), axis=1, keepdims=True)
            new_comps.append(jnp.where(lane_iota == j, v, comps[p]))
        arow = assl_ref[pl.ds(row, 1), :]
        av = jnp.sum(jnp.where(onehot, arow, 0.0), axis=1, keepdims=True)
        assv = jnp.where(lane_iota == j, av, assv)
        actfg = jnp.where(lane_iota == j, flg_ref[j], actfg)
        return tuple(new_comps), assv, actfg

    comps0 = tuple(zero_row for _ in range(8))
    comps, assv, actfg = jax.lax.fori_loop(
        0, _ROIS, gather_body, (comps0, zero_row, zero_row))

    exl1, eyl1, exl2, eyl2, exr1, eyr1, exr2, eyr2 = comps

    gsel = [jnp.zeros((1, _LANES), jnp.float32) for _ in range(20)]
    for g in range(32):
        grow = gt_ref[g:g + 1, :]
        sel = assv == jnp.float32(g)
        for c in range(20):
            gsel[c] = jnp.where(sel, grow[:, c:c + 1], gsel[c])
    glx1, gly1, glx2, gly2 = gsel[0], gsel[1], gsel[2], gsel[3]
    grx1, gry1, grx2, gry2 = gsel[4], gsel[5], gsel[6], gsel[7]
    glabel = gsel[8]
    dims = gsel[9:14]
    kpts = gsel[14:20]

    labels = jnp.where(actfg > 0.0, glabel, 0.0)
    fg3 = labels > 0.0
    lab1 = labels == 1.0

    dxl, dyl, dwl, dhl = _transform(exl1, eyl1, exl2, eyl2, glx1, gly1, glx2, gly2)
    dxr, dyr, dwr, dhr = _transform(exr1, eyr1, exr2, eyr2, grx1, gry1, grx2, gry2)
    bbox_stds = (0.1, 0.1, 0.2, 0.2)
    tl = [jnp.where(fg3, t / s, 0.0) for t, s in zip((dxl, dyl, dwl, dhl), bbox_stds)]
    tr = [jnp.where(fg3, t / s, 0.0) for t, s in zip((dxr, dyr, dwr, dhr), bbox_stds)]

    dim_means = (1.52, 1.62, 3.88, 0.0, 0.0)
    dim_stds = (0.42, 0.38, 1.35, 1.0, 1.0)
    dimt = [jnp.where(fg3, (d - m) / s, 0.0)
            for d, m, s in zip(dims, dim_means, dim_stds)]

    start = exl1
    width = exl2 - exl1 + 1.0
    tk = []
    for c in range(6):
        t = jnp.round((kpts[c] - start) * _KPTS_GRID / width)
        t = jnp.where((t < 0.0) | (t > _KPTS_GRID - 1.0), -225.0, t)
        tk.append(t)
    pos = tk[0]
    typ = jnp.zeros((1, _LANES), jnp.float32)
    for c in range(1, 4):
        u = tk[c] > pos
        typ = jnp.where(u, jnp.float32(c), typ)
        pos = jnp.where(u, tk[c], pos)
    kt = [typ * _KPTS_GRID + pos, tk[4], tk[5]]
    kw = [jnp.where(t < 0.0, 0.0, 1.0) for t in kt]
    kt = [jnp.where(t < 0.0, 0.0, t) for t in kt]
    kt = [jnp.where(lab1, t, 0.0) for t in kt]
    kw = [jnp.where(lab1, w, 0.0) for w in kw]

    ins = [jnp.where(fg3, 1.0, 0.0) for _ in range(4)]

    bvec = jnp.zeros((1, _LANES), jnp.float32) + pl.program_id(0).astype(jnp.float32)
    rows = ([bvec, exl1, eyl1, exl2, eyl2]
            + [bvec, exr1, eyr1, exr2, eyr2]
            + [labels]
            + tl + tr + dimt + kt + kw + ins + ins
            + [zero_row, zero_row])
    out_ref[...] = jnp.concatenate(rows, axis=0)


def kernel(all_rois_left, all_rois_right, gt_boxes_left, gt_boxes_right,
           gt_dim_orien, gt_kpts, num_boxes):
    B = all_rois_left.shape[0]
    nreal = all_rois_left.shape[1] + gt_boxes_left.shape[1]

    gt_app_l = jnp.zeros_like(gt_boxes_left).at[:, :, 1:5].set(gt_boxes_left[:, :, :4])
    gt_app_r = jnp.zeros_like(gt_boxes_right).at[:, :, 1:5].set(gt_boxes_right[:, :, :4])
    all_l = jnp.concatenate([all_rois_left, gt_app_l], axis=1)
    all_r = jnp.concatenate([all_rois_right, gt_app_r], axis=1)

    boxes = jnp.concatenate([all_l[:, :, 1:5], all_r[:, :, 1:5]], axis=2)
    boxes = jnp.transpose(boxes, (0, 2, 1))
    boxes = jnp.pad(boxes, ((0, 0), (0, 0), (0, _NP - nreal)))
    boxes = boxes.reshape(B, 8, _ROWS, _LANES)

    gt = jnp.concatenate([gt_boxes_left[:, :, :4], gt_boxes_right[:, :, :4],
                          gt_boxes_left[:, :, 4:5], gt_dim_orien, gt_kpts], axis=2)
    gt = jnp.pad(gt, ((0, 0), (0, 0), (0, _LANES - gt.shape[2])))

    nreal_arr = jnp.full((1,), nreal, jnp.int32)

    out = pl.pallas_call(
        _ptl_kernel,
        grid=(B,),
        in_specs=[
            pl.BlockSpec((None, 8, _ROWS, _LANES), lambda b: (b, 0, 0, 0)),
            pl.BlockSpec((None, 32, _LANES), lambda b: (b, 0, 0)),
            pl.BlockSpec(memory_space=pltpu.SMEM),
        ],
        out_specs=pl.BlockSpec((None, 40, _LANES), lambda b: (b, 0, 0)),
        out_shape=jax.ShapeDtypeStruct((B, 40, _LANES), jnp.float32),
        scratch_shapes=[pltpu.VMEM((_ROWS, _LANES), jnp.float32),
                        pltpu.SMEM((_ROIS,), jnp.int32),
                        pltpu.SMEM((_ROIS,), jnp.float32)],
        compiler_params=pltpu.CompilerParams(
            dimension_semantics=("parallel",)),
    )(boxes, gt, nreal_arr)

    rois_l = jnp.transpose(out[:, 0:5, :], (0, 2, 1))
    rois_r = jnp.transpose(out[:, 5:10, :], (0, 2, 1))
    labels = out[:, 10, :]
    btl = jnp.transpose(out[:, 11:15, :], (0, 2, 1))
    btr = jnp.transpose(out[:, 15:19, :], (0, 2, 1))
    dimt = jnp.transpose(out[:, 19:24, :], (0, 2, 1))
    kt = jnp.transpose(out[:, 24:27, :], (0, 2, 1)).astype(jnp.int64)
    kw = jnp.transpose(out[:, 27:30, :], (0, 2, 1))
    ins = jnp.transpose(out[:, 30:34, :], (0, 2, 1))
    outs = jnp.transpose(out[:, 34:38, :], (0, 2, 1))
    return rois_l, rois_r, labels, btl, btr, dimt, kt, kw, ins, outs
